# Initial kernel scaffold; baseline (speedup 1.0000x reference)
import jax
import jax.numpy as jnp
from jax.experimental import pallas as pl


def kernel(x, edge_index, edge_attr, W1, as1, ad1, We1, ae1, b1, W2, as2, ad2, We2, ae2, b2, Wm1, bm1, Wm2, bm2):
    raise NotImplementedError("write your pallas kernel here")


# speedup vs baseline: 10.8042x; 10.8042x over previous
import dataclasses
import functools

import jax
import jax.numpy as jnp
from jax import lax
from jax.experimental import pallas as pl
from jax.experimental.pallas import tpu as pltpu
from jax.experimental.pallas import tpu_sc as plsc

N = 10000
E = 320000
D = 128
C = 64
ED = 16

NP = 10240
NCORE = 2
NSUB = 16
NW = NCORE * NSUB
NPS = NP // NSUB
NPW = NP // NW
EPW = E // NW
EPWP = 10240
EP = NW * EPWP
ROWS = EP // 128
RPW = ROWS // NW
PAD_NODE = NP - 1

_MESH = plsc.VectorSubcoreMesh(core_axis_name="c", subcore_axis_name="s",
                               num_cores=NCORE, num_subcores=NSUB)

_SC_PARAMS = pltpu.CompilerParams(needs_layout_passes=False,
                                  use_tc_tiling_on_sc=False)

_HIGH = jax.lax.Precision.HIGHEST


def _dot(a, b):
    return jax.lax.dot_general(a, b, (((1,), (0,)), ((), ())),
                               precision=_HIGH,
                               preferred_element_type=jnp.float32)


def _lrelu(x):
    return jnp.maximum(x, 0.2 * x)


def _tc_node_pre(xpad, W1, as1, ad1):
    BN = 1024

    def body(x_ref, w_ref, as_ref, ad_ref, xp_ref, a_ref, d_ref):
        xp = _dot(x_ref[...], w_ref[...])
        xp_ref[...] = xp
        a_ref[...] = _dot(xp, as_ref[...])
        d_ref[...] = _dot(xp, ad_ref[...])

    return pl.pallas_call(
        body,
        grid=(NP // BN,),
        in_specs=[
            pl.BlockSpec((BN, D), lambda i: (i, 0)),
            pl.BlockSpec((D, C), lambda i: (0, 0)),
            pl.BlockSpec((C, 1), lambda i: (0, 0)),
            pl.BlockSpec((C, 1), lambda i: (0, 0)),
        ],
        out_specs=[
            pl.BlockSpec((BN, C), lambda i: (i, 0)),
            pl.BlockSpec((BN, 1), lambda i: (i, 0)),
            pl.BlockSpec((BN, 1), lambda i: (i, 0)),
        ],
        out_shape=[
            jax.ShapeDtypeStruct((NP, C), jnp.float32),
            jax.ShapeDtypeStruct((NP, 1), jnp.float32),
            jax.ShapeDtypeStruct((NP, 1), jnp.float32),
        ],
    )(xpad, W1, as1.reshape(C, 1), ad1.reshape(C, 1))


def _tc_edge_pre(eaP, We1, ae1, We2, ae2, Wm1b, bm1):
    BE = 4096

    def body(ea_ref, we1_ref, ae1_ref, we2_ref, ae2_ref, wmb_ref, bm1_ref,
             g1_ref, g2_ref, r_ref):
        ea = ea_ref[...]
        v1 = _dot(we1_ref[...], ae1_ref[...])
        v2 = _dot(we2_ref[...], ae2_ref[...])
        g1_ref[...] = _dot(ea, v1)
        g2_ref[...] = _dot(ea, v2)
        r_ref[...] = _dot(ea, wmb_ref[...]) + bm1_ref[...]

    return pl.pallas_call(
        body,
        grid=(EP // BE,),
        in_specs=[
            pl.BlockSpec((BE, ED), lambda i: (i, 0)),
            pl.BlockSpec((ED, C), lambda i: (0, 0)),
            pl.BlockSpec((C, 1), lambda i: (0, 0)),
            pl.BlockSpec((ED, C), lambda i: (0, 0)),
            pl.BlockSpec((C, 1), lambda i: (0, 0)),
            pl.BlockSpec((ED, C), lambda i: (0, 0)),
            pl.BlockSpec((1, C), lambda i: (0, 0)),
        ],
        out_specs=[
            pl.BlockSpec((BE, 1), lambda i: (i, 0)),
            pl.BlockSpec((BE, 1), lambda i: (i, 0)),
            pl.BlockSpec((BE, C), lambda i: (i, 0)),
        ],
        out_shape=[
            jax.ShapeDtypeStruct((EP, 1), jnp.float32),
            jax.ShapeDtypeStruct((EP, 1), jnp.float32),
            jax.ShapeDtypeStruct((EP, C), jnp.float32),
        ],
    )(eaP, We1, ae1.reshape(C, 1), We2, ae2.reshape(C, 1), Wm1b,
      bm1.reshape(1, C))


def _tc_node_mid(acc0, acc1, sw, xp, b, Wa, Wb):
    BN = 1024

    def body(a0_ref, a1_ref, sw_ref, xp_ref, b_ref, wa_ref, wb_ref,
             oa_ref, ob_ref):
        h = a0_ref[...] + a1_ref[...] + sw_ref[...] * xp_ref[...] + b_ref[...]
        h = jnp.maximum(h, 0.0)
        oa_ref[...] = _dot(h, wa_ref[...])
        ob_ref[...] = _dot(h, wb_ref[...])

    return pl.pallas_call(
        body,
        grid=(NP // BN,),
        in_specs=[
            pl.BlockSpec((BN, C), lambda i: (i, 0)),
            pl.BlockSpec((BN, C), lambda i: (i, 0)),
            pl.BlockSpec((BN, 1), lambda i: (i, 0)),
            pl.BlockSpec((BN, C), lambda i: (i, 0)),
            pl.BlockSpec((1, C), lambda i: (0, 0)),
            pl.BlockSpec((C, C), lambda i: (0, 0)),
            pl.BlockSpec((C, C), lambda i: (0, 0)),
        ],
        out_specs=[
            pl.BlockSpec((BN, C), lambda i: (i, 0)),
            pl.BlockSpec((BN, C), lambda i: (i, 0)),
        ],
        out_shape=[
            jax.ShapeDtypeStruct((NP, C), jnp.float32),
            jax.ShapeDtypeStruct((NP, C), jnp.float32),
        ],
    )(acc0, acc1, sw, xp, b.reshape(1, C), Wa, Wb)


def _tc_node_layer(acc0, acc1, sw, xp, b, W, as_, ad_):
    BN = 1024

    def body(a0_ref, a1_ref, sw_ref, xp_ref, b_ref, w_ref, as_ref, ad_ref,
             xp2_ref, a_ref, d_ref):
        h = a0_ref[...] + a1_ref[...] + sw_ref[...] * xp_ref[...] + b_ref[...]
        h = jnp.maximum(h, 0.0)
        xp2 = _dot(h, w_ref[...])
        xp2_ref[...] = xp2
        a_ref[...] = _dot(xp2, as_ref[...])
        d_ref[...] = _dot(xp2, ad_ref[...])

    return pl.pallas_call(
        body,
        grid=(NP // BN,),
        in_specs=[
            pl.BlockSpec((BN, C), lambda i: (i, 0)),
            pl.BlockSpec((BN, C), lambda i: (i, 0)),
            pl.BlockSpec((BN, 1), lambda i: (i, 0)),
            pl.BlockSpec((BN, C), lambda i: (i, 0)),
            pl.BlockSpec((1, C), lambda i: (0, 0)),
            pl.BlockSpec((C, C), lambda i: (0, 0)),
            pl.BlockSpec((C, 1), lambda i: (0, 0)),
            pl.BlockSpec((C, 1), lambda i: (0, 0)),
        ],
        out_specs=[
            pl.BlockSpec((BN, C), lambda i: (i, 0)),
            pl.BlockSpec((BN, 1), lambda i: (i, 0)),
            pl.BlockSpec((BN, 1), lambda i: (i, 0)),
        ],
        out_shape=[
            jax.ShapeDtypeStruct((NP, C), jnp.float32),
            jax.ShapeDtypeStruct((NP, 1), jnp.float32),
            jax.ShapeDtypeStruct((NP, 1), jnp.float32),
        ],
    )(acc0, acc1, sw, xp, b.reshape(1, C), W, as_.reshape(C, 1),
      ad_.reshape(C, 1))


def _tc_vecdots(xp, as_, ad_):
    BN = 1024

    def body(xp_ref, as_ref, ad_ref, a_ref, d_ref):
        a_ref[...] = _dot(xp_ref[...], as_ref[...])
        d_ref[...] = _dot(xp_ref[...], ad_ref[...])

    return pl.pallas_call(
        body,
        grid=(NP // BN,),
        in_specs=[
            pl.BlockSpec((BN, C), lambda i: (i, 0)),
            pl.BlockSpec((C, 1), lambda i: (0, 0)),
            pl.BlockSpec((C, 1), lambda i: (0, 0)),
        ],
        out_specs=[
            pl.BlockSpec((BN, 1), lambda i: (i, 0)),
            pl.BlockSpec((BN, 1), lambda i: (i, 0)),
        ],
        out_shape=[
            jax.ShapeDtypeStruct((NP, 1), jnp.float32),
            jax.ShapeDtypeStruct((NP, 1), jnp.float32),
        ],
    )(xp, as_.reshape(C, 1), ad_.reshape(C, 1))


def _worker_id():
    return lax.axis_index("c") * NSUB + lax.axis_index("s")


def _sc_pass1_first(aT, dT, g1P, g2P, srcP, dstP, kv):
    @functools.partial(
        pl.kernel,
        out_type=(
            jax.ShapeDtypeStruct((ROWS, 128), jnp.float32),
            jax.ShapeDtypeStruct((NCORE * NP,), jnp.float32),
            jax.ShapeDtypeStruct((NCORE * NP,), jnp.float32),
            jax.ShapeDtypeStruct((NCORE * NP,), jnp.float32),
            jax.ShapeDtypeStruct((NCORE * NP,), jnp.float32),
        ),
        mesh=_MESH,
        compiler_params=_SC_PARAMS,
        scratch_types=[
            pltpu.VMEM((NP,), jnp.float32),
            pltpu.VMEM((NP,), jnp.float32),
            pltpu.VMEM((16,), jnp.float32),
            pltpu.VMEM((16, 128), jnp.int32),
            pltpu.VMEM((16, 128), jnp.int32),
            pltpu.VMEM((16, 128), jnp.float32),
            pltpu.VMEM((16, 128), jnp.float32),
            pltpu.VMEM((16, 128), jnp.float32),
            pltpu.VMEM((16, 128), jnp.float32),
            pltpu.VMEM((NPS,), jnp.float32),
            pltpu.VMEM_SHARED((NP,), jnp.float32),
            pltpu.VMEM_SHARED((NP,), jnp.float32),
            pltpu.VMEM_SHARED((NP,), jnp.float32),
            pltpu.VMEM_SHARED((NP,), jnp.float32),
        ],
    )
    def k(aT_h, dT_h, g1_h, g2_h, src_h, dst_h, kv_h,
          ex_o, den_o, deg_o, sg1_o, sg2_o,
          ta, td, kvv, srcv, dstv, g1v, g2v, exv, onesv, zv,
          dens, degs, sg1s, sg2s):
        c = lax.axis_index("c")
        s = lax.axis_index("s")
        w = c * NSUB + s

        @pl.loop(0, NPS, step=16)
        def _(i):
            zv[pl.ds(i, 16)] = jnp.zeros((16,), jnp.float32)

        @pl.loop(0, 16)
        def _(r):
            @pl.loop(0, 128, step=16)
            def _(i):
                onesv[r, pl.ds(i, 16)] = jnp.ones((16,), jnp.float32)

        ns = s * NPS
        pltpu.sync_copy(zv, dens.at[pl.ds(ns, NPS)])
        pltpu.sync_copy(zv, degs.at[pl.ds(ns, NPS)])
        pltpu.sync_copy(zv, sg1s.at[pl.ds(ns, NPS)])
        pltpu.sync_copy(zv, sg2s.at[pl.ds(ns, NPS)])
        pltpu.sync_copy(aT_h, ta)
        pltpu.sync_copy(dT_h, td)
        pltpu.sync_copy(kv_h, kvv)
        plsc.subcore_barrier()
        K = kvv[...]

        row0 = w * RPW

        @pl.loop(0, RPW, step=16)
        def _(r0):
            rb = row0 + r0
            pltpu.sync_copy(src_h.at[pl.ds(rb, 16)], srcv)
            pltpu.sync_copy(dst_h.at[pl.ds(rb, 16)], dstv)
            pltpu.sync_copy(g1_h.at[pl.ds(rb, 16)], g1v)
            pltpu.sync_copy(g2_h.at[pl.ds(rb, 16)], g2v)

            @pl.loop(0, 16)
            def _(j):
                @pl.loop(0, 128, step=16)
                def _(i):
                    si = srcv[j, pl.ds(i, 16)]
                    di = dstv[j, pl.ds(i, 16)]
                    av = plsc.load_gather(ta, [si])
                    dv = plsc.load_gather(td, [di])
                    al = _lrelu(av + dv + g1v[j, pl.ds(i, 16)])
                    exv[j, pl.ds(i, 16)] = jnp.exp(al - K)

            pltpu.sync_copy(exv, ex_o.at[pl.ds(rb, 16)])

            @pl.loop(0, 16)
            def _(j):
                pltpu.sync_copy(exv.at[j], dens.at[dstv.at[j]], add=True)
                pltpu.sync_copy(onesv.at[j], degs.at[dstv.at[j]], add=True)
                pltpu.sync_copy(g1v.at[j], sg1s.at[dstv.at[j]], add=True)
                pltpu.sync_copy(g2v.at[j], sg2s.at[dstv.at[j]], add=True)

        plsc.subcore_barrier()
        sl = pl.ds(ns, NPS)
        osl = pl.ds(c * NP + ns, NPS)
        pltpu.sync_copy(dens.at[sl], den_o.at[osl])
        pltpu.sync_copy(degs.at[sl], deg_o.at[osl])
        pltpu.sync_copy(sg1s.at[sl], sg1_o.at[osl])
        pltpu.sync_copy(sg2s.at[sl], sg2_o.at[osl])

    return k(aT, dT, g1P, g2P, srcP, dstP, kv)


def _sc_pass1_second(aT, dT, gP, srcP, dstP, kv):
    @functools.partial(
        pl.kernel,
        out_type=(
            jax.ShapeDtypeStruct((ROWS, 128), jnp.float32),
            jax.ShapeDtypeStruct((NCORE * NP,), jnp.float32),
        ),
        mesh=_MESH,
        compiler_params=_SC_PARAMS,
        scratch_types=[
            pltpu.VMEM((NP,), jnp.float32),
            pltpu.VMEM((NP,), jnp.float32),
            pltpu.VMEM((16,), jnp.float32),
            pltpu.VMEM((16, 128), jnp.int32),
            pltpu.VMEM((16, 128), jnp.int32),
            pltpu.VMEM((16, 128), jnp.float32),
            pltpu.VMEM((16, 128), jnp.float32),
            pltpu.VMEM((NPS,), jnp.float32),
            pltpu.VMEM_SHARED((NP,), jnp.float32),
        ],
    )
    def k(aT_h, dT_h, g_h, src_h, dst_h, kv_h,
          ex_o, den_o,
          ta, td, kvv, srcv, dstv, gv, exv, zv, dens):
        c = lax.axis_index("c")
        s = lax.axis_index("s")
        w = c * NSUB + s

        @pl.loop(0, NPS, step=16)
        def _(i):
            zv[pl.ds(i, 16)] = jnp.zeros((16,), jnp.float32)

        ns = s * NPS
        pltpu.sync_copy(zv, dens.at[pl.ds(ns, NPS)])
        pltpu.sync_copy(aT_h, ta)
        pltpu.sync_copy(dT_h, td)
        pltpu.sync_copy(kv_h, kvv)
        plsc.subcore_barrier()
        K = kvv[...]

        row0 = w * RPW

        @pl.loop(0, RPW, step=16)
        def _(r0):
            rb = row0 + r0
            pltpu.sync_copy(src_h.at[pl.ds(rb, 16)], srcv)
            pltpu.sync_copy(dst_h.at[pl.ds(rb, 16)], dstv)
            pltpu.sync_copy(g_h.at[pl.ds(rb, 16)], gv)

            @pl.loop(0, 16)
            def _(j):
                @pl.loop(0, 128, step=16)
                def _(i):
                    si = srcv[j, pl.ds(i, 16)]
                    di = dstv[j, pl.ds(i, 16)]
                    av = plsc.load_gather(ta, [si])
                    dv = plsc.load_gather(td, [di])
                    al = _lrelu(av + dv + gv[j, pl.ds(i, 16)])
                    exv[j, pl.ds(i, 16)] = jnp.exp(al - K)

            pltpu.sync_copy(exv, ex_o.at[pl.ds(rb, 16)])

            @pl.loop(0, 16)
            def _(j):
                pltpu.sync_copy(exv.at[j], dens.at[dstv.at[j]], add=True)

        plsc.subcore_barrier()
        sl = pl.ds(ns, NPS)
        pltpu.sync_copy(dens.at[sl], den_o.at[pl.ds(c * NP + ns, NPS)])

    return k(aT, dT, gP, srcP, dstP, kv)


def _sc_combine(aT, dT, den_p, deg_p, sg_p, kv):
    @functools.partial(
        pl.kernel,
        out_type=(
            jax.ShapeDtypeStruct((NP,), jnp.float32),
            jax.ShapeDtypeStruct((NP,), jnp.float32),
        ),
        mesh=_MESH,
        compiler_params=_SC_PARAMS,
        scratch_types=[
            pltpu.VMEM((NPW,), jnp.float32),
            pltpu.VMEM((NPW,), jnp.float32),
            pltpu.VMEM((NPW,), jnp.float32),
            pltpu.VMEM((NPW,), jnp.float32),
            pltpu.VMEM((NPW,), jnp.float32),
            pltpu.VMEM((NPW,), jnp.float32),
            pltpu.VMEM((NPW,), jnp.float32),
            pltpu.VMEM((NPW,), jnp.float32),
            pltpu.VMEM((NPW,), jnp.float32),
            pltpu.VMEM((NPW,), jnp.float32),
            pltpu.VMEM((16,), jnp.float32),
        ],
    )
    def k(aT_h, dT_h, denp_h, degp_h, sgp_h, kv_h,
          den_o, sw_o,
          av, dv, den0, den1, deg0, deg1, sg0, sg1, deno, swo, kvv):
        c = lax.axis_index("c")
        s = lax.axis_index("s")
        w = c * NSUB + s
        nb = w * NPW
        sl = pl.ds(nb, NPW)
        pltpu.sync_copy(aT_h.at[sl], av)
        pltpu.sync_copy(dT_h.at[sl], dv)
        pltpu.sync_copy(denp_h.at[pl.ds(nb, NPW)], den0)
        pltpu.sync_copy(denp_h.at[pl.ds(NP + nb, NPW)], den1)
        pltpu.sync_copy(degp_h.at[pl.ds(nb, NPW)], deg0)
        pltpu.sync_copy(degp_h.at[pl.ds(NP + nb, NPW)], deg1)
        pltpu.sync_copy(sgp_h.at[pl.ds(nb, NPW)], sg0)
        pltpu.sync_copy(sgp_h.at[pl.ds(NP + nb, NPW)], sg1)
        pltpu.sync_copy(kv_h, kvv)
        K = kvv[...]

        @pl.loop(0, NPW, step=16)
        def _(i):
            g = pl.ds(i, 16)
            deg = deg0[g] + deg1[g]
            sg = sg0[g] + sg1[g]
            als = _lrelu(av[g] + dv[g] + sg / jnp.maximum(deg, 1.0))
            exs = jnp.exp(als - K)
            den = den0[g] + den1[g] + exs
            deno[g] = den
            swo[g] = exs / den

        pltpu.sync_copy(deno, den_o.at[sl])
        pltpu.sync_copy(swo, sw_o.at[sl])

    return k(aT, dT, den_p, deg_p, sg_p, kv)


def _sc_pass2(exP, den, srcP, dstP, xp):
    @functools.partial(
        pl.kernel,
        out_type=jax.ShapeDtypeStruct((NCORE * NP, C), jnp.float32),
        mesh=_MESH,
        compiler_params=_SC_PARAMS,
        scratch_types=[
            pltpu.VMEM((NP,), jnp.float32),
            pltpu.VMEM((16, 128), jnp.int32),
            pltpu.VMEM((16, 128), jnp.int32),
            pltpu.VMEM((16, 128), jnp.float32),
            pltpu.VMEM((128, C), jnp.float32),
            pltpu.VMEM((128,), jnp.float32),
            pltpu.VMEM((64, C), jnp.float32),
            pltpu.VMEM_SHARED((NP, C), jnp.float32),
        ],
    )
    def k(ex_h, den_h, src_h, dst_h, xp_h,
          acc_o,
          tden, srcv, dstv, exv, rows, wv, zv, accs):
        c = lax.axis_index("c")
        s = lax.axis_index("s")
        w = c * NSUB + s

        @pl.loop(0, 64)
        def _(r):
            @pl.loop(0, C, step=16)
            def _(i):
                zv[r, pl.ds(i, 16)] = jnp.zeros((16,), jnp.float32)

        ns = s * NPS

        @pl.loop(0, NPS, step=64)
        def _(i):
            pltpu.sync_copy(zv, accs.at[pl.ds(ns + i, 64)])

        pltpu.sync_copy(den_h, tden)
        plsc.subcore_barrier()

        row0 = w * RPW

        @pl.loop(0, RPW, step=16)
        def _(r0):
            rb = row0 + r0
            pltpu.sync_copy(src_h.at[pl.ds(rb, 16)], srcv)
            pltpu.sync_copy(dst_h.at[pl.ds(rb, 16)], dstv)
            pltpu.sync_copy(ex_h.at[pl.ds(rb, 16)], exv)

            @pl.loop(0, 16)
            def _(j):
                pltpu.sync_copy(xp_h.at[srcv.at[j]], rows)

                @pl.loop(0, 128, step=16)
                def _(i):
                    dvi = plsc.load_gather(tden, [dstv[j, pl.ds(i, 16)]])
                    wv[pl.ds(i, 16)] = exv[j, pl.ds(i, 16)] / dvi

                @pl.loop(0, 128)
                def _(e):
                    esplat = jnp.zeros((16,), jnp.int32) + e
                    we = plsc.load_gather(wv, [esplat])
                    for cb in range(C // 16):
                        g = pl.ds(cb * 16, 16)
                        rows[e, g] = rows[e, g] * we

                pltpu.sync_copy(rows, accs.at[dstv.at[j]], add=True)

        plsc.subcore_barrier()
        sl = pl.ds(ns, NPS)
        pltpu.sync_copy(accs.at[sl], acc_o.at[pl.ds(c * NP + ns, NPS)])

    return k(exP, den, srcP, dstP, xp)


def _sc_final(Pt, Qt, RP, srcP, dstP, wm2, b2v):
    @functools.partial(
        pl.kernel,
        out_type=jax.ShapeDtypeStruct((ROWS, 128), jnp.float32),
        mesh=_MESH,
        compiler_params=_SC_PARAMS,
        scratch_types=[
            pltpu.VMEM((16, 128), jnp.int32),
            pltpu.VMEM((16, 128), jnp.int32),
            pltpu.VMEM((128, C), jnp.float32),
            pltpu.VMEM((128, C), jnp.float32),
            pltpu.VMEM((128, C), jnp.float32),
            pltpu.VMEM((C,), jnp.float32),
            pltpu.VMEM((16,), jnp.float32),
            pltpu.VMEM((16, 16), jnp.float32),
            pltpu.VMEM((16, 128), jnp.float32),
        ],
    )
    def k(p_h, q_h, r_h, src_h, dst_h, wm2_h, b2_h,
          out_o,
          srcv, dstv, prow, qrow, rrow, tw, bv, part, outv):
        c = lax.axis_index("c")
        s = lax.axis_index("s")
        w = c * NSUB + s
        pltpu.sync_copy(wm2_h, tw)
        pltpu.sync_copy(b2_h, bv)
        m0 = tw[pl.ds(0, 16)]
        m1 = tw[pl.ds(16, 16)]
        m2 = tw[pl.ds(32, 16)]
        m3 = tw[pl.ds(48, 16)]
        bias = bv[...]
        riota = lax.iota(jnp.int32, 16)

        row0 = w * RPW

        @pl.loop(0, RPW, step=16)
        def _(r0):
            rb = row0 + r0
            pltpu.sync_copy(src_h.at[pl.ds(rb, 16)], srcv)
            pltpu.sync_copy(dst_h.at[pl.ds(rb, 16)], dstv)

            @pl.loop(0, 16)
            def _(j):
                pltpu.sync_copy(p_h.at[srcv.at[j]], prow)
                pltpu.sync_copy(q_h.at[dstv.at[j]], qrow)
                pltpu.sync_copy(r_h.at[pl.ds((rb + j) * 128, 128)], rrow)

                @pl.loop(0, 128, step=16)
                def _(i):
                    @pl.loop(0, 16)
                    def _(e2):
                        e = i + e2
                        g0 = pl.ds(0, 16)
                        g1 = pl.ds(16, 16)
                        g2 = pl.ds(32, 16)
                        g3 = pl.ds(48, 16)
                        t0 = jnp.maximum(prow[e, g0] + qrow[e, g0] + rrow[e, g0], 0.0)
                        t1 = jnp.maximum(prow[e, g1] + qrow[e, g1] + rrow[e, g1], 0.0)
                        t2 = jnp.maximum(prow[e, g2] + qrow[e, g2] + rrow[e, g2], 0.0)
                        t3 = jnp.maximum(prow[e, g3] + qrow[e, g3] + rrow[e, g3], 0.0)
                        part[e2, :] = t0 * m0 + t1 * m1 + t2 * m2 + t3 * m3

                    acc = bias

                    def col(l, a):
                        cv = plsc.load_gather(part, [riota, jnp.full((16,), l, jnp.int32)])
                        return a + cv

                    acc = lax.fori_loop(0, 16, col, acc)
                    outv[j, pl.ds(i, 16)] = acc

            pltpu.sync_copy(outv, out_o.at[pl.ds(rb, 16)])

    return k(Pt, Qt, RP, srcP, dstP, wm2, b2v)


def kernel(x, edge_index, edge_attr, W1, as1, ad1, We1, ae1, b1,
           W2, as2, ad2, We2, ae2, b2, Wm1, bm1, Wm2, bm2):
    src = edge_index[0]
    dst = edge_index[1]

    xpad = jnp.pad(x, ((0, NP - N), (0, 0)))
    srcP = jnp.pad(src.reshape(NW, EPW), ((0, 0), (0, EPWP - EPW)),
                   constant_values=0).reshape(ROWS, 128)
    dstP = jnp.pad(dst.reshape(NW, EPW), ((0, 0), (0, EPWP - EPW)),
                   constant_values=PAD_NODE).reshape(ROWS, 128)
    eaP = jnp.pad(edge_attr.reshape(NW, EPW, ED), ((0, 0), (0, EPWP - EPW), (0, 0))
                  ).reshape(EP, ED)

    xp1, a1, d1 = _tc_node_pre(xpad, W1, as1, ad1)
    g1, g2, RP = _tc_edge_pre(eaP, We1, ae1, We2, ae2, Wm1[C:C + ED], bm1)
    a1f = a1.reshape(NP)
    d1f = d1.reshape(NP)
    g1P = g1.reshape(ROWS, 128)
    g2P = g2.reshape(ROWS, 128)

    K1 = _lrelu(jnp.max(a1f) + jnp.max(d1f) + jnp.maximum(jnp.max(g1), 0.0))
    kv1 = jnp.full((16,), K1, jnp.float32)

    ex1, den1p, degp, sg1p, sg2p = _sc_pass1_first(
        a1f, d1f, g1P, g2P, srcP, dstP, kv1)
    den1, sw1 = _sc_combine(a1f, d1f, den1p, degp, sg1p, kv1)
    accf1 = _sc_pass2(ex1, den1, srcP, dstP, xp1)

    xp2, a2, d2 = _tc_node_layer(accf1[:NP], accf1[NP:], sw1.reshape(NP, 1),
                                 xp1, b1, W2, as2, ad2)
    a2f = a2.reshape(NP)
    d2f = d2.reshape(NP)
    K2 = _lrelu(jnp.max(a2f) + jnp.max(d2f) + jnp.maximum(jnp.max(g2), 0.0))
    kv2 = jnp.full((16,), K2, jnp.float32)

    ex2, den2p = _sc_pass1_second(a2f, d2f, g2P, srcP, dstP, kv2)
    den2, sw2 = _sc_combine(a2f, d2f, den2p, degp, sg2p, kv2)
    accf2 = _sc_pass2(ex2, den2, srcP, dstP, xp2)

    Pt, Qt = _tc_node_mid(accf2[:NP], accf2[NP:], sw2.reshape(NP, 1), xp2,
                          b2, Wm1[:C], Wm1[C + ED:])

    b2v = jnp.full((16,), bm2[0], jnp.float32)
    outP = _sc_final(Pt, Qt, RP, srcP, dstP, Wm2.reshape(C), b2v)

    out = outP.reshape(NW, EPWP)[:, :EPW].reshape(E, 1)
    return out


# speedup vs baseline: 11.1774x; 1.0345x over previous
import functools

import jax
import jax.numpy as jnp
from jax import lax
from jax.experimental import pallas as pl
from jax.experimental.pallas import tpu as pltpu
from jax.experimental.pallas import tpu_sc as plsc

N = 10000
E = 320000
D = 128
C = 64
ED = 16

NP = 10240
NCORE = 2
NSUB = 16
NW = NCORE * NSUB
NPS = NP // NSUB
NPW = NP // NW
EPW = E // NW
EPWP = 10240
EP = NW * EPWP
ROWS = EP // 128
RPW = ROWS // NW
PAD_NODE = NP - 1
NCHK = 2560

_MESH = plsc.VectorSubcoreMesh(core_axis_name="c", subcore_axis_name="s",
                               num_cores=NCORE, num_subcores=NSUB)

_SC_PARAMS = pltpu.CompilerParams(needs_layout_passes=False,
                                  use_tc_tiling_on_sc=False)

_HIGH = jax.lax.Precision.HIGHEST


def _dot(a, b):
    return jax.lax.dot_general(a, b, (((1,), (0,)), ((), ())),
                               precision=_HIGH,
                               preferred_element_type=jnp.float32)


def _lrelu(x):
    return jnp.maximum(x, 0.2 * x)


def _tc_node_pre(xpad, W1, as1, ad1):
    BN = 1024

    def body(x_ref, w_ref, as_ref, ad_ref, xp_ref, a_ref, d_ref):
        xp = _dot(x_ref[...], w_ref[...])
        xp_ref[...] = xp
        a_ref[...] = _dot(xp, as_ref[...])
        d_ref[...] = _dot(xp, ad_ref[...])

    return pl.pallas_call(
        body,
        grid=(NP // BN,),
        in_specs=[
            pl.BlockSpec((BN, D), lambda i: (i, 0)),
            pl.BlockSpec((D, C), lambda i: (0, 0)),
            pl.BlockSpec((C, 1), lambda i: (0, 0)),
            pl.BlockSpec((C, 1), lambda i: (0, 0)),
        ],
        out_specs=[
            pl.BlockSpec((BN, C), lambda i: (i, 0)),
            pl.BlockSpec((BN, 1), lambda i: (i, 0)),
            pl.BlockSpec((BN, 1), lambda i: (i, 0)),
        ],
        out_shape=[
            jax.ShapeDtypeStruct((NP, C), jnp.float32),
            jax.ShapeDtypeStruct((NP, 1), jnp.float32),
            jax.ShapeDtypeStruct((NP, 1), jnp.float32),
        ],
    )(xpad, W1, as1.reshape(C, 1), ad1.reshape(C, 1))


def _tc_edge_g(eaP, We1, ae1, We2, ae2):
    BE = 8192

    def body(ea_ref, we1_ref, ae1_ref, we2_ref, ae2_ref, g1_ref, g2_ref):
        ea = ea_ref[...]
        v1 = _dot(we1_ref[...], ae1_ref[...])
        v2 = _dot(we2_ref[...], ae2_ref[...])
        g1_ref[...] = _dot(ea, v1)
        g2_ref[...] = _dot(ea, v2)

    return pl.pallas_call(
        body,
        grid=(EP // BE,),
        in_specs=[
            pl.BlockSpec((BE, ED), lambda i: (i, 0)),
            pl.BlockSpec((ED, C), lambda i: (0, 0)),
            pl.BlockSpec((C, 1), lambda i: (0, 0)),
            pl.BlockSpec((ED, C), lambda i: (0, 0)),
            pl.BlockSpec((C, 1), lambda i: (0, 0)),
        ],
        out_specs=[
            pl.BlockSpec((BE, 1), lambda i: (i, 0)),
            pl.BlockSpec((BE, 1), lambda i: (i, 0)),
        ],
        out_shape=[
            jax.ShapeDtypeStruct((EP, 1), jnp.float32),
            jax.ShapeDtypeStruct((EP, 1), jnp.float32),
        ],
    )(eaP, We1, ae1.reshape(C, 1), We2, ae2.reshape(C, 1))


def _tc_edge_r(eaP, Wm1b, bm1):
    BE = 8192

    def body(ea_ref, wmb_ref, bm1_ref, r_ref):
        r_ref[...] = _dot(ea_ref[...], wmb_ref[...]) + bm1_ref[...]

    return pl.pallas_call(
        body,
        grid=(EP // BE,),
        in_specs=[
            pl.BlockSpec((BE, ED), lambda i: (i, 0)),
            pl.BlockSpec((ED, C), lambda i: (0, 0)),
            pl.BlockSpec((1, C), lambda i: (0, 0)),
        ],
        out_specs=[
            pl.BlockSpec((BE, C), lambda i: (i, 0)),
        ],
        out_shape=[
            jax.ShapeDtypeStruct((EP, C), jnp.float32),
        ],
    )(eaP, Wm1b, bm1.reshape(1, C))[0]


def _tc_node_mid(acc0, acc1, sw, xp, b, Wa, Wb):
    BN = 1024

    def body(a0_ref, a1_ref, sw_ref, xp_ref, b_ref, wa_ref, wb_ref,
             oa_ref, ob_ref):
        h = a0_ref[...] + a1_ref[...] + sw_ref[...] * xp_ref[...] + b_ref[...]
        h = jnp.maximum(h, 0.0)
        oa_ref[...] = _dot(h, wa_ref[...])
        ob_ref[...] = _dot(h, wb_ref[...])

    return pl.pallas_call(
        body,
        grid=(NP // BN,),
        in_specs=[
            pl.BlockSpec((BN, C), lambda i: (i, 0)),
            pl.BlockSpec((BN, C), lambda i: (i, 0)),
            pl.BlockSpec((BN, 1), lambda i: (i, 0)),
            pl.BlockSpec((BN, C), lambda i: (i, 0)),
            pl.BlockSpec((1, C), lambda i: (0, 0)),
            pl.BlockSpec((C, C), lambda i: (0, 0)),
            pl.BlockSpec((C, C), lambda i: (0, 0)),
        ],
        out_specs=[
            pl.BlockSpec((BN, C), lambda i: (i, 0)),
            pl.BlockSpec((BN, C), lambda i: (i, 0)),
        ],
        out_shape=[
            jax.ShapeDtypeStruct((NP, C), jnp.float32),
            jax.ShapeDtypeStruct((NP, C), jnp.float32),
        ],
    )(acc0, acc1, sw, xp, b.reshape(1, C), Wa, Wb)


def _tc_node_layer(acc0, acc1, sw, xp, b, W, as_, ad_):
    BN = 1024

    def body(a0_ref, a1_ref, sw_ref, xp_ref, b_ref, w_ref, as_ref, ad_ref,
             xp2_ref, a_ref, d_ref):
        h = a0_ref[...] + a1_ref[...] + sw_ref[...] * xp_ref[...] + b_ref[...]
        h = jnp.maximum(h, 0.0)
        xp2 = _dot(h, w_ref[...])
        xp2_ref[...] = xp2
        a_ref[...] = _dot(xp2, as_ref[...])
        d_ref[...] = _dot(xp2, ad_ref[...])

    return pl.pallas_call(
        body,
        grid=(NP // BN,),
        in_specs=[
            pl.BlockSpec((BN, C), lambda i: (i, 0)),
            pl.BlockSpec((BN, C), lambda i: (i, 0)),
            pl.BlockSpec((BN, 1), lambda i: (i, 0)),
            pl.BlockSpec((BN, C), lambda i: (i, 0)),
            pl.BlockSpec((1, C), lambda i: (0, 0)),
            pl.BlockSpec((C, C), lambda i: (0, 0)),
            pl.BlockSpec((C, 1), lambda i: (0, 0)),
            pl.BlockSpec((C, 1), lambda i: (0, 0)),
        ],
        out_specs=[
            pl.BlockSpec((BN, C), lambda i: (i, 0)),
            pl.BlockSpec((BN, 1), lambda i: (i, 0)),
            pl.BlockSpec((BN, 1), lambda i: (i, 0)),
        ],
        out_shape=[
            jax.ShapeDtypeStruct((NP, C), jnp.float32),
            jax.ShapeDtypeStruct((NP, 1), jnp.float32),
            jax.ShapeDtypeStruct((NP, 1), jnp.float32),
        ],
    )(acc0, acc1, sw, xp, b.reshape(1, C), W, as_.reshape(C, 1),
      ad_.reshape(C, 1))


def _sc_pass1_first(aT, dT, g1F, g2F, srcF, dstF, kv):
    @functools.partial(
        pl.kernel,
        out_type=(
            jax.ShapeDtypeStruct((EP,), jnp.float32),
            jax.ShapeDtypeStruct((NCORE * NP,), jnp.float32),
            jax.ShapeDtypeStruct((NCORE * NP,), jnp.float32),
            jax.ShapeDtypeStruct((NCORE * NP,), jnp.float32),
            jax.ShapeDtypeStruct((NCORE * NP,), jnp.float32),
        ),
        mesh=_MESH,
        compiler_params=_SC_PARAMS,
        scratch_types=[
            pltpu.VMEM((NP,), jnp.float32),
            pltpu.VMEM((NP,), jnp.float32),
            pltpu.VMEM((16,), jnp.float32),
            pltpu.VMEM((EPWP,), jnp.int32),
            pltpu.VMEM((EPWP,), jnp.int32),
            pltpu.VMEM((EPWP,), jnp.float32),
            pltpu.VMEM((EPWP,), jnp.float32),
            pltpu.VMEM((EPWP,), jnp.float32),
            pltpu.VMEM((EPWP,), jnp.float32),
            pltpu.VMEM((NPS,), jnp.float32),
            pltpu.VMEM_SHARED((NP,), jnp.float32),
            pltpu.VMEM_SHARED((NP,), jnp.float32),
            pltpu.VMEM_SHARED((NP,), jnp.float32),
            pltpu.VMEM_SHARED((NP,), jnp.float32),
        ],
    )
    def k(aT_h, dT_h, g1_h, g2_h, src_h, dst_h, kv_h,
          ex_o, den_o, deg_o, sg1_o, sg2_o,
          ta, td, kvv, srcv, dstv, g1v, g2v, exv, onesv, zv,
          dens, degs, sg1s, sg2s):
        c = lax.axis_index("c")
        s = lax.axis_index("s")
        w = c * NSUB + s

        @pl.loop(0, NPS, step=16)
        def _(i):
            zv[pl.ds(i, 16)] = jnp.zeros((16,), jnp.float32)

        @pl.loop(0, EPWP, step=16)
        def _(i):
            onesv[pl.ds(i, 16)] = jnp.ones((16,), jnp.float32)

        ns = s * NPS
        pltpu.sync_copy(zv, dens.at[pl.ds(ns, NPS)])
        pltpu.sync_copy(zv, degs.at[pl.ds(ns, NPS)])
        pltpu.sync_copy(zv, sg1s.at[pl.ds(ns, NPS)])
        pltpu.sync_copy(zv, sg2s.at[pl.ds(ns, NPS)])
        eb = w * EPWP
        esl = pl.ds(eb, EPWP)
        pltpu.sync_copy(aT_h, ta)
        pltpu.sync_copy(dT_h, td)
        pltpu.sync_copy(kv_h, kvv)
        pltpu.sync_copy(src_h.at[esl], srcv)
        pltpu.sync_copy(dst_h.at[esl], dstv)
        pltpu.sync_copy(g1_h.at[esl], g1v)
        pltpu.sync_copy(g2_h.at[esl], g2v)
        plsc.subcore_barrier()
        K = kvv[...]

        @pl.loop(0, EPWP, step=16)
        def _(i):
            g = pl.ds(i, 16)
            av = plsc.load_gather(ta, [srcv[g]])
            dv = plsc.load_gather(td, [dstv[g]])
            al = _lrelu(av + dv + g1v[g])
            exv[g] = jnp.exp(al - K)

        pltpu.sync_copy(exv, ex_o.at[esl])
        pltpu.sync_copy(exv, dens.at[dstv], add=True)
        pltpu.sync_copy(onesv, degs.at[dstv], add=True)
        pltpu.sync_copy(g1v, sg1s.at[dstv], add=True)
        pltpu.sync_copy(g2v, sg2s.at[dstv], add=True)

        plsc.subcore_barrier()
        sl = pl.ds(ns, NPS)
        osl = pl.ds(c * NP + ns, NPS)
        pltpu.sync_copy(dens.at[sl], den_o.at[osl])
        pltpu.sync_copy(degs.at[sl], deg_o.at[osl])
        pltpu.sync_copy(sg1s.at[sl], sg1_o.at[osl])
        pltpu.sync_copy(sg2s.at[sl], sg2_o.at[osl])

    return k(aT, dT, g1F, g2F, srcF, dstF, kv)


def _sc_pass1_second(aT, dT, gF, srcF, dstF, kv):
    @functools.partial(
        pl.kernel,
        out_type=(
            jax.ShapeDtypeStruct((EP,), jnp.float32),
            jax.ShapeDtypeStruct((NCORE * NP,), jnp.float32),
        ),
        mesh=_MESH,
        compiler_params=_SC_PARAMS,
        scratch_types=[
            pltpu.VMEM((NP,), jnp.float32),
            pltpu.VMEM((NP,), jnp.float32),
            pltpu.VMEM((16,), jnp.float32),
            pltpu.VMEM((EPWP,), jnp.int32),
            pltpu.VMEM((EPWP,), jnp.int32),
            pltpu.VMEM((EPWP,), jnp.float32),
            pltpu.VMEM((EPWP,), jnp.float32),
            pltpu.VMEM((NPS,), jnp.float32),
            pltpu.VMEM_SHARED((NP,), jnp.float32),
        ],
    )
    def k(aT_h, dT_h, g_h, src_h, dst_h, kv_h,
          ex_o, den_o,
          ta, td, kvv, srcv, dstv, gv, exv, zv, dens):
        c = lax.axis_index("c")
        s = lax.axis_index("s")
        w = c * NSUB + s

        @pl.loop(0, NPS, step=16)
        def _(i):
            zv[pl.ds(i, 16)] = jnp.zeros((16,), jnp.float32)

        ns = s * NPS
        pltpu.sync_copy(zv, dens.at[pl.ds(ns, NPS)])
        eb = w * EPWP
        esl = pl.ds(eb, EPWP)
        pltpu.sync_copy(aT_h, ta)
        pltpu.sync_copy(dT_h, td)
        pltpu.sync_copy(kv_h, kvv)
        pltpu.sync_copy(src_h.at[esl], srcv)
        pltpu.sync_copy(dst_h.at[esl], dstv)
        pltpu.sync_copy(g_h.at[esl], gv)
        plsc.subcore_barrier()
        K = kvv[...]

        @pl.loop(0, EPWP, step=16)
        def _(i):
            g = pl.ds(i, 16)
            av = plsc.load_gather(ta, [srcv[g]])
            dv = plsc.load_gather(td, [dstv[g]])
            al = _lrelu(av + dv + gv[g])
            exv[g] = jnp.exp(al - K)

        pltpu.sync_copy(exv, ex_o.at[esl])
        pltpu.sync_copy(exv, dens.at[dstv], add=True)

        plsc.subcore_barrier()
        sl = pl.ds(ns, NPS)
        pltpu.sync_copy(dens.at[sl], den_o.at[pl.ds(c * NP + ns, NPS)])

    return k(aT, dT, gF, srcF, dstF, kv)


def _sc_pass2(exF, den_p, deg_p, sg_p, aT, dT, kv, srcP, dstP, xp):
    @functools.partial(
        pl.kernel,
        out_type=(
            jax.ShapeDtypeStruct((NCORE * NP, C), jnp.float32),
            jax.ShapeDtypeStruct((NP,), jnp.float32),
        ),
        mesh=_MESH,
        compiler_params=_SC_PARAMS,
        scratch_types=[
            pltpu.VMEM((NP,), jnp.float32),
            pltpu.VMEM((NCHK,), jnp.float32),
            pltpu.VMEM((NCHK,), jnp.float32),
            pltpu.VMEM((NCHK,), jnp.float32),
            pltpu.VMEM((NCHK,), jnp.float32),
            pltpu.VMEM((NCHK,), jnp.float32),
            pltpu.VMEM((NCHK,), jnp.float32),
            pltpu.VMEM((NCHK,), jnp.float32),
            pltpu.VMEM((NCHK,), jnp.float32),
            pltpu.VMEM((NPW,), jnp.float32),
            pltpu.VMEM((16,), jnp.float32),
            pltpu.VMEM((EPWP,), jnp.float32),
            pltpu.VMEM((RPW, 128), jnp.int32),
            pltpu.VMEM((RPW, 128), jnp.int32),
            pltpu.VMEM((128, C), jnp.float32),
            pltpu.VMEM((128,), jnp.float32),
            pltpu.VMEM((64, C), jnp.float32),
            pltpu.VMEM_SHARED((NP, C), jnp.float32),
        ],
    )
    def k(ex_h, denp_h, degp_h, sgp_h, aT_h, dT_h, kv_h, src_h, dst_h, xp_h,
          acc_o, sw_o,
          tden, p0, p1, p2, p3, p4, p5, p6, p7, swv, kvv,
          exv, srcv, dstv, rows, wv, zv, accs):
        c = lax.axis_index("c")
        s = lax.axis_index("s")
        w = c * NSUB + s

        pltpu.sync_copy(kv_h, kvv)
        K = kvv[...]

        @pl.loop(0, NP, step=NCHK)
        def _(nb0):
            pltpu.sync_copy(denp_h.at[pl.ds(nb0, NCHK)], p0)
            pltpu.sync_copy(denp_h.at[pl.ds(NP + nb0, NCHK)], p1)
            pltpu.sync_copy(degp_h.at[pl.ds(nb0, NCHK)], p2)
            pltpu.sync_copy(degp_h.at[pl.ds(NP + nb0, NCHK)], p3)
            pltpu.sync_copy(sgp_h.at[pl.ds(nb0, NCHK)], p4)
            pltpu.sync_copy(sgp_h.at[pl.ds(NP + nb0, NCHK)], p5)
            pltpu.sync_copy(aT_h.at[pl.ds(nb0, NCHK)], p6)
            pltpu.sync_copy(dT_h.at[pl.ds(nb0, NCHK)], p7)

            @pl.loop(0, NCHK, step=16)
            def _(i):
                g = pl.ds(i, 16)
                deg = p2[g] + p3[g]
                sg = p4[g] + p5[g]
                als = _lrelu(p6[g] + p7[g] + sg / jnp.maximum(deg, 1.0))
                exs = jnp.exp(als - K)
                tden[pl.ds(nb0 + i, 16)] = p0[g] + p1[g] + exs

        nb = w * NPW
        pltpu.sync_copy(denp_h.at[pl.ds(nb, NPW)], p0.at[pl.ds(0, NPW)])
        pltpu.sync_copy(denp_h.at[pl.ds(NP + nb, NPW)], p1.at[pl.ds(0, NPW)])
        pltpu.sync_copy(degp_h.at[pl.ds(nb, NPW)], p2.at[pl.ds(0, NPW)])
        pltpu.sync_copy(degp_h.at[pl.ds(NP + nb, NPW)], p3.at[pl.ds(0, NPW)])
        pltpu.sync_copy(sgp_h.at[pl.ds(nb, NPW)], p4.at[pl.ds(0, NPW)])
        pltpu.sync_copy(sgp_h.at[pl.ds(NP + nb, NPW)], p5.at[pl.ds(0, NPW)])
        pltpu.sync_copy(aT_h.at[pl.ds(nb, NPW)], p6.at[pl.ds(0, NPW)])
        pltpu.sync_copy(dT_h.at[pl.ds(nb, NPW)], p7.at[pl.ds(0, NPW)])

        @pl.loop(0, NPW, step=16)
        def _(i):
            g = pl.ds(i, 16)
            deg = p2[g] + p3[g]
            sg = p4[g] + p5[g]
            als = _lrelu(p6[g] + p7[g] + sg / jnp.maximum(deg, 1.0))
            exs = jnp.exp(als - K)
            swv[g] = exs / tden[pl.ds(nb + i, 16)]

        pltpu.sync_copy(swv, sw_o.at[pl.ds(nb, NPW)])

        @pl.loop(0, 64)
        def _(r):
            @pl.loop(0, C, step=16)
            def _(i):
                zv[r, pl.ds(i, 16)] = jnp.zeros((16,), jnp.float32)

        ns = s * NPS

        @pl.loop(0, NPS, step=64)
        def _(i):
            pltpu.sync_copy(zv, accs.at[pl.ds(ns + i, 64)])

        eb = w * EPWP
        pltpu.sync_copy(ex_h.at[pl.ds(eb, EPWP)], exv)
        row0 = w * RPW
        pltpu.sync_copy(src_h.at[pl.ds(row0, RPW)], srcv)
        pltpu.sync_copy(dst_h.at[pl.ds(row0, RPW)], dstv)
        plsc.subcore_barrier()

        @pl.loop(0, RPW)
        def _(j):
            pltpu.sync_copy(xp_h.at[srcv.at[j]], rows)

            @pl.loop(0, 128, step=16)
            def _(i):
                dvi = plsc.load_gather(tden, [dstv[j, pl.ds(i, 16)]])
                wv[pl.ds(i, 16)] = exv[pl.ds(j * 128 + i, 16)] / dvi

            @pl.loop(0, 128)
            def _(e):
                esplat = jnp.zeros((16,), jnp.int32) + e
                we = plsc.load_gather(wv, [esplat])
                for cb in range(C // 16):
                    g = pl.ds(cb * 16, 16)
                    rows[e, g] = rows[e, g] * we

            pltpu.sync_copy(rows, accs.at[dstv.at[j]], add=True)

        plsc.subcore_barrier()
        sl = pl.ds(ns, NPS)
        pltpu.sync_copy(accs.at[sl], acc_o.at[pl.ds(c * NP + ns, NPS)])

    return k(exF, den_p, deg_p, sg_p, aT, dT, kv, srcP, dstP, xp)


def _sc_final(Pt, Qt, RP, srcP, dstP, wm2, b2v):
    @functools.partial(
        pl.kernel,
        out_type=jax.ShapeDtypeStruct((ROWS, 128), jnp.float32),
        mesh=_MESH,
        compiler_params=_SC_PARAMS,
        scratch_types=[
            pltpu.VMEM((RPW, 128), jnp.int32),
            pltpu.VMEM((RPW, 128), jnp.int32),
            pltpu.VMEM((128, C), jnp.float32),
            pltpu.VMEM((128, C), jnp.float32),
            pltpu.VMEM((128, C), jnp.float32),
            pltpu.VMEM((C,), jnp.float32),
            pltpu.VMEM((16,), jnp.float32),
            pltpu.VMEM((16, 16), jnp.float32),
            pltpu.VMEM((RPW, 128), jnp.float32),
        ],
    )
    def k(p_h, q_h, r_h, src_h, dst_h, wm2_h, b2_h,
          out_o,
          srcv, dstv, prow, qrow, rrow, tw, bv, part, outv):
        c = lax.axis_index("c")
        s = lax.axis_index("s")
        w = c * NSUB + s
        pltpu.sync_copy(wm2_h, tw)
        pltpu.sync_copy(b2_h, bv)
        row0 = w * RPW
        pltpu.sync_copy(src_h.at[pl.ds(row0, RPW)], srcv)
        pltpu.sync_copy(dst_h.at[pl.ds(row0, RPW)], dstv)
        m0 = tw[pl.ds(0, 16)]
        m1 = tw[pl.ds(16, 16)]
        m2 = tw[pl.ds(32, 16)]
        m3 = tw[pl.ds(48, 16)]
        bias = bv[...]
        riota = lax.iota(jnp.int32, 16)

        @pl.loop(0, RPW)
        def _(j):
            pltpu.sync_copy(p_h.at[srcv.at[j]], prow)
            pltpu.sync_copy(q_h.at[dstv.at[j]], qrow)
            pltpu.sync_copy(r_h.at[pl.ds((row0 + j) * 128, 128)], rrow)

            @pl.loop(0, 128, step=16)
            def _(i):
                @pl.loop(0, 16)
                def _(e2):
                    e = i + e2
                    g0 = pl.ds(0, 16)
                    g1 = pl.ds(16, 16)
                    g2 = pl.ds(32, 16)
                    g3 = pl.ds(48, 16)
                    t0 = jnp.maximum(prow[e, g0] + qrow[e, g0] + rrow[e, g0], 0.0)
                    t1 = jnp.maximum(prow[e, g1] + qrow[e, g1] + rrow[e, g1], 0.0)
                    t2 = jnp.maximum(prow[e, g2] + qrow[e, g2] + rrow[e, g2], 0.0)
                    t3 = jnp.maximum(prow[e, g3] + qrow[e, g3] + rrow[e, g3], 0.0)
                    part[e2, :] = t0 * m0 + t1 * m1 + t2 * m2 + t3 * m3

                acc = bias

                def col(l, a):
                    cv = plsc.load_gather(part, [riota, jnp.full((16,), l, jnp.int32)])
                    return a + cv

                acc = lax.fori_loop(0, 16, col, acc)
                outv[j, pl.ds(i, 16)] = acc

        pltpu.sync_copy(outv, out_o.at[pl.ds(row0, RPW)])

    return k(Pt, Qt, RP, srcP, dstP, wm2, b2v)


def kernel(x, edge_index, edge_attr, W1, as1, ad1, We1, ae1, b1,
           W2, as2, ad2, We2, ae2, b2, Wm1, bm1, Wm2, bm2):
    src = edge_index[0]
    dst = edge_index[1]

    xpad = jnp.pad(x, ((0, NP - N), (0, 0)))
    srcF = jnp.pad(src.reshape(NW, EPW), ((0, 0), (0, EPWP - EPW)),
                   constant_values=0).reshape(EP)
    dstF = jnp.pad(dst.reshape(NW, EPW), ((0, 0), (0, EPWP - EPW)),
                   constant_values=PAD_NODE).reshape(EP)
    srcP = srcF.reshape(ROWS, 128)
    dstP = dstF.reshape(ROWS, 128)
    eaP = jnp.pad(edge_attr.reshape(NW, EPW, ED), ((0, 0), (0, EPWP - EPW), (0, 0))
                  ).reshape(EP, ED)

    xp1, a1, d1 = _tc_node_pre(xpad, W1, as1, ad1)
    g1, g2 = _tc_edge_g(eaP, We1, ae1, We2, ae2)
    RP = _tc_edge_r(eaP, Wm1[C:C + ED], bm1)
    a1f = a1.reshape(NP)
    d1f = d1.reshape(NP)
    g1F = g1.reshape(EP)
    g2F = g2.reshape(EP)

    K1 = _lrelu(jnp.max(a1f) + jnp.max(d1f) + jnp.maximum(jnp.max(g1F), 0.0))
    kv1 = jnp.full((16,), K1, jnp.float32)

    ex1, den1p, degp, sg1p, sg2p = _sc_pass1_first(
        a1f, d1f, g1F, g2F, srcF, dstF, kv1)
    accf1, sw1 = _sc_pass2(ex1, den1p, degp, sg1p, a1f, d1f, kv1,
                           srcP, dstP, xp1)

    xp2, a2, d2 = _tc_node_layer(accf1[:NP], accf1[NP:], sw1.reshape(NP, 1),
                                 xp1, b1, W2, as2, ad2)
    a2f = a2.reshape(NP)
    d2f = d2.reshape(NP)
    K2 = _lrelu(jnp.max(a2f) + jnp.max(d2f) + jnp.maximum(jnp.max(g2F), 0.0))
    kv2 = jnp.full((16,), K2, jnp.float32)

    ex2, den2p = _sc_pass1_second(a2f, d2f, g2F, srcF, dstF, kv2)
    accf2, sw2 = _sc_pass2(ex2, den2p, degp, sg2p, a2f, d2f, kv2,
                           srcP, dstP, xp2)

    Pt, Qt = _tc_node_mid(accf2[:NP], accf2[NP:], sw2.reshape(NP, 1), xp2,
                          b2, Wm1[:C], Wm1[C + ED:])

    b2v = jnp.full((16,), bm2[0], jnp.float32)
    outP = _sc_final(Pt, Qt, RP, srcP, dstP, Wm2.reshape(C), b2v)

    out = outP.reshape(NW, EPWP)[:, :EPW].reshape(E, 1)
    return out
